# baseline (speedup 1.0000x reference)
import functools

import jax
import jax.numpy as jnp
from jax import lax
from jax.experimental import pallas as pl
from jax.experimental.pallas import tpu as pltpu
from jax.experimental.pallas import tpu_sc as plsc

E_OLD = 640000
E_KEPT = 320000
D = 128

NC = 2
NS = 16
NW = NC * NS
CPW = E_OLD // NW
CCH = 5000
KPW = E_KEPT // NW
SCAT = 80
NCHUNK = KPW // SCAT
BLK = 5
NBLK = NCHUNK // BLK
KB = BLK * SCAT

_mesh = plsc.VectorSubcoreMesh(core_axis_name="c", subcore_axis_name="s")


def _copy_body(old_hbm, out_hbm, sem):
    wid = lax.axis_index("s") * NC + lax.axis_index("c")
    base = wid * CPW
    handles = []
    for j in range(CPW // CCH):
        off = pl.multiple_of(base + j * CCH, 8)
        handles.append(pltpu.async_copy(
            old_hbm.at[pl.ds(off, CCH)], out_hbm.at[pl.ds(off, CCH)], sem))
    for h in handles:
        h.wait()


_copy = functools.partial(
    pl.kernel,
    mesh=_mesh,
    out_type=jax.ShapeDtypeStruct((E_OLD, D), jnp.float32),
    scratch_types=[pltpu.SemaphoreType.DMA],
)(_copy_body)


def _scatter_body(mask3_hbm, kept_hbm, out_hbm, idx_v, kept_buf, sem_s):
    wid = lax.axis_index("s") * NC + lax.axis_index("c")
    pltpu.sync_copy(mask3_hbm.at[wid], idx_v)

    def scatter_block(b, _):
        off = pl.multiple_of(wid * KPW + b * KB, 8)
        pltpu.sync_copy(kept_hbm.at[pl.ds(off, KB)], kept_buf)
        handles = []
        for c in range(BLK):
            handles.append(pltpu.async_copy(
                kept_buf.at[pl.ds(c * SCAT, SCAT)],
                out_hbm.at[idx_v.at[b * BLK + c]],
                sem_s))
        for h in handles:
            h.wait()
        return 0

    lax.fori_loop(0, NBLK, scatter_block, 0)


_scatter = functools.partial(
    pl.kernel,
    mesh=_mesh,
    out_type=(),
    scratch_types=[
        pltpu.VMEM((NCHUNK, SCAT), jnp.int32),
        pltpu.VMEM((KB, D), jnp.float32),
        pltpu.SemaphoreType.DMA,
    ],
)(_scatter_body)


def kernel(old_features, mask_idx, kept_features, edge_index_old):
    mask3 = mask_idx.reshape(NW, NCHUNK, SCAT)
    copied = _copy(old_features)
    out_ref = jax.new_ref(copied)
    _scatter(mask3, kept_features, out_ref)
    return out_ref[...], edge_index_old


# speedup vs baseline: 28.0143x; 28.0143x over previous
import functools

import jax
import jax.numpy as jnp
from jax import lax
from jax.experimental import pallas as pl
from jax.experimental.pallas import tpu as pltpu
from jax.experimental.pallas import tpu_sc as plsc

E_OLD = 640000
E_KEPT = 320000
D = 128

NC = 2
NS = 16
NW = NC * NS
CPW = E_OLD // NW
CCH = 5000
KPW = E_KEPT // NW
SCAT = 80
NCHUNK = KPW // SCAT
BLK = 5
NBLK = NCHUNK // BLK
KB = BLK * SCAT

_mesh = plsc.VectorSubcoreMesh(core_axis_name="c", subcore_axis_name="s")


COPY_ROWS = 6400


def _copy_body(old_ref, out_ref):
    out_ref[...] = old_ref[...]


def _copy(old_features):
    return pl.pallas_call(
        _copy_body,
        grid=(E_OLD // COPY_ROWS,),
        in_specs=[pl.BlockSpec((COPY_ROWS, D), lambda i: (i, 0))],
        out_specs=pl.BlockSpec((COPY_ROWS, D), lambda i: (i, 0)),
        out_shape=jax.ShapeDtypeStruct((E_OLD, D), jnp.float32),
    )(old_features)


def _scatter_body(mask3_hbm, kept_hbm, out_hbm, idx_v, kept_buf, sem_s):
    wid = lax.axis_index("s") * NC + lax.axis_index("c")
    pltpu.sync_copy(mask3_hbm.at[wid], idx_v)

    def scatter_block(b, _):
        off = pl.multiple_of(wid * KPW + b * KB, 8)
        pltpu.sync_copy(kept_hbm.at[pl.ds(off, KB)], kept_buf)
        handles = []
        for c in range(BLK):
            handles.append(pltpu.async_copy(
                kept_buf.at[pl.ds(c * SCAT, SCAT)],
                out_hbm.at[idx_v.at[b * BLK + c]],
                sem_s))
        for h in handles:
            h.wait()
        return 0

    lax.fori_loop(0, NBLK, scatter_block, 0)


_scatter = functools.partial(
    pl.kernel,
    mesh=_mesh,
    out_type=(),
    scratch_types=[
        pltpu.VMEM((NCHUNK, SCAT), jnp.int32),
        pltpu.VMEM((KB, D), jnp.float32),
        pltpu.SemaphoreType.DMA,
    ],
)(_scatter_body)


def kernel(old_features, mask_idx, kept_features, edge_index_old):
    mask3 = mask_idx.reshape(NW, NCHUNK, SCAT)
    copied = _copy(old_features)
    out_ref = jax.new_ref(copied)
    _scatter(mask3, kept_features, out_ref)
    return out_ref[...], edge_index_old


# speedup vs baseline: 29.1788x; 1.0416x over previous
import functools

import jax
import jax.numpy as jnp
from jax import lax
from jax.experimental import pallas as pl
from jax.experimental.pallas import tpu as pltpu
from jax.experimental.pallas import tpu_sc as plsc

E_OLD = 640000
E_KEPT = 320000
D = 128

NC = 2
NS = 16
NW = NC * NS
CPW = E_OLD // NW
CCH = 5000
KPW = E_KEPT // NW
SCAT = 80
NCHUNK = KPW // SCAT
BLK = 5
NBLK = NCHUNK // BLK
KB = BLK * SCAT

_mesh = plsc.VectorSubcoreMesh(core_axis_name="c", subcore_axis_name="s")


COPY_ROWS = 6400


def _copy_body(old_ref, out_ref):
    out_ref[...] = old_ref[...]


def _copy(old_features):
    return pl.pallas_call(
        _copy_body,
        grid=(E_OLD // COPY_ROWS,),
        in_specs=[pl.BlockSpec((COPY_ROWS, D), lambda i: (i, 0))],
        out_specs=pl.BlockSpec((COPY_ROWS, D), lambda i: (i, 0)),
        out_shape=jax.ShapeDtypeStruct((E_OLD, D), jnp.float32),
    )(old_features)


def _scatter_body(mask3_hbm, kept_hbm, out_hbm, idx_v, buf0, buf1,
                  sem_l, sem_s):
    wid = lax.axis_index("s") * NC + lax.axis_index("c")
    pltpu.sync_copy(mask3_hbm.at[wid], idx_v)

    def kept_at(b):
        off = pl.multiple_of(wid * KPW + lax.min(b, NBLK - 1) * KB, 8)
        return kept_hbm.at[pl.ds(off, KB)]

    def fire(buf, b):
        return [pltpu.async_copy(buf.at[pl.ds(c * SCAT, SCAT)],
                                 out_hbm.at[idx_v.at[b * BLK + c]], sem_s)
                for c in range(BLK)]

    pltpu.async_copy(kept_at(0), buf0, sem_l)

    def pair(g, _):
        b0 = 2 * g
        pltpu.make_async_copy(kept_at(b0), buf0, sem_l).wait()
        h1 = pltpu.async_copy(kept_at(b0 + 1), buf1, sem_l)
        s0 = fire(buf0, b0)
        h1.wait()
        for h in s0:
            h.wait()
        pltpu.async_copy(kept_at(b0 + 2), buf0, sem_l)
        s1 = fire(buf1, b0 + 1)
        for h in s1:
            h.wait()
        return 0

    lax.fori_loop(0, NBLK // 2, pair, 0)
    last = NBLK - 1
    pltpu.make_async_copy(kept_at(last), buf0, sem_l).wait()
    for h in fire(buf0, last):
        h.wait()


_scatter = functools.partial(
    pl.kernel,
    mesh=_mesh,
    out_type=(),
    scratch_types=[
        pltpu.VMEM((NCHUNK, SCAT), jnp.int32),
        pltpu.VMEM((KB, D), jnp.float32),
        pltpu.VMEM((KB, D), jnp.float32),
        pltpu.SemaphoreType.DMA,
        pltpu.SemaphoreType.DMA,
    ],
)(_scatter_body)


def kernel(old_features, mask_idx, kept_features, edge_index_old):
    mask3 = mask_idx.reshape(NW, NCHUNK, SCAT)
    copied = _copy(old_features)
    out_ref = jax.new_ref(copied)
    _scatter(mask3, kept_features, out_ref)
    return out_ref[...], edge_index_old


# speedup vs baseline: 29.4895x; 1.0106x over previous
import functools

import jax
import jax.numpy as jnp
from jax import lax
from jax.experimental import pallas as pl
from jax.experimental.pallas import tpu as pltpu
from jax.experimental.pallas import tpu_sc as plsc

E_OLD = 640000
E_KEPT = 320000
D = 128

NC = 2
NS = 16
NW = NC * NS
CPW = E_OLD // NW
CCH = 5000
KPW = E_KEPT // NW
SCAT = 80
NCHUNK = KPW // SCAT
BLK = 5
NBLK = NCHUNK // BLK
KB = BLK * SCAT

_mesh = plsc.VectorSubcoreMesh(core_axis_name="c", subcore_axis_name="s")


COPY_ROWS = 12800


def _copy_body(old_ref, out_ref):
    out_ref[...] = old_ref[...]


def _copy(old_features):
    return pl.pallas_call(
        _copy_body,
        grid=(E_OLD // COPY_ROWS,),
        in_specs=[pl.BlockSpec((COPY_ROWS, D), lambda i: (i, 0))],
        out_specs=pl.BlockSpec((COPY_ROWS, D), lambda i: (i, 0)),
        out_shape=jax.ShapeDtypeStruct((E_OLD, D), jnp.float32),
    )(old_features)


def _scatter_body(mask3_hbm, kept_hbm, out_hbm, idx_v, buf0, buf1,
                  sem_l, sem_s):
    wid = lax.axis_index("s") * NC + lax.axis_index("c")
    pltpu.sync_copy(mask3_hbm.at[wid], idx_v)

    def kept_at(b):
        off = pl.multiple_of(wid * KPW + lax.min(b, NBLK - 1) * KB, 8)
        return kept_hbm.at[pl.ds(off, KB)]

    def fire(buf, b):
        return [pltpu.async_copy(buf.at[pl.ds(c * SCAT, SCAT)],
                                 out_hbm.at[idx_v.at[b * BLK + c]], sem_s)
                for c in range(BLK)]

    pltpu.async_copy(kept_at(0), buf0, sem_l)

    def pair(g, _):
        b0 = 2 * g
        pltpu.make_async_copy(kept_at(b0), buf0, sem_l).wait()
        h1 = pltpu.async_copy(kept_at(b0 + 1), buf1, sem_l)
        s0 = fire(buf0, b0)
        h1.wait()
        for h in s0:
            h.wait()
        pltpu.async_copy(kept_at(b0 + 2), buf0, sem_l)
        s1 = fire(buf1, b0 + 1)
        for h in s1:
            h.wait()
        return 0

    lax.fori_loop(0, NBLK // 2, pair, 0)
    last = NBLK - 1
    pltpu.make_async_copy(kept_at(last), buf0, sem_l).wait()
    for h in fire(buf0, last):
        h.wait()


_scatter = functools.partial(
    pl.kernel,
    mesh=_mesh,
    out_type=(),
    scratch_types=[
        pltpu.VMEM((NCHUNK, SCAT), jnp.int32),
        pltpu.VMEM((KB, D), jnp.float32),
        pltpu.VMEM((KB, D), jnp.float32),
        pltpu.SemaphoreType.DMA,
        pltpu.SemaphoreType.DMA,
    ],
)(_scatter_body)


def kernel(old_features, mask_idx, kept_features, edge_index_old):
    mask3 = mask_idx.reshape(NW, NCHUNK, SCAT)
    copied = _copy(old_features)
    out_ref = jax.new_ref(copied)
    _scatter(mask3, kept_features, out_ref)
    return out_ref[...], edge_index_old
